# baseline (speedup 1.0000x reference)
import functools

import jax
import jax.numpy as jnp
from jax import lax
from jax.experimental import pallas as pl
from jax.experimental.pallas import tpu as pltpu
from jax.experimental.pallas import tpu_sc as plsc

_B = 16
_T = 16384
_D = 512
_VP = 1024
_NW = 32
_CH = _T // _NW
_HBINS = 3 * _B * _VP


def _hist_body(tq_hbm, sq_hbm, ts_hbm, ss_hbm, tv_hbm, sv_hbm, z_hbm, out_hbm,
               tq_v, sq_v, ts_v, ss_v, tv_v, sv_v, hist_v, sem):
    wid = lax.axis_index("s") * 2 + lax.axis_index("c")
    base = wid * _CH

    cps = [
        pltpu.async_copy(z_hbm, hist_v, sem),
        pltpu.async_copy(tq_hbm.at[pl.ds(base, _CH)], tq_v, sem),
        pltpu.async_copy(sq_hbm.at[pl.ds(base, _CH)], sq_v, sem),
        pltpu.async_copy(ts_hbm.at[pl.ds(base, _CH)], ts_v, sem),
        pltpu.async_copy(ss_hbm.at[pl.ds(base, _CH)], ss_v, sem),
        pltpu.async_copy(tv_hbm.at[pl.ds(base, _CH)], tv_v, sem),
        pltpu.async_copy(sv_hbm.at[pl.ds(base, _CH)], sv_v, sem),
    ]
    for c in cps:
        c.wait()

    ones16 = jnp.ones((16,), jnp.float32)
    for s, (tok_v, seg_v) in enumerate(
            ((tq_v, sq_v), (ts_v, ss_v), (tv_v, sv_v))):
        s_off = s * _B * _VP
        for i in range(_CH // 16):
            tok = tok_v[pl.ds(i * 16, 16)]
            seg = seg_v[pl.ds(i * 16, 16)]
            idx = seg * _VP + tok + s_off
            plsc.addupdate_scatter(hist_v, [idx], ones16)

    pltpu.sync_copy(hist_v, out_hbm.at[wid])


@functools.cache
def _make_sc_hist():
    return pl.kernel(
        _hist_body,
        mesh=plsc.VectorSubcoreMesh(core_axis_name="c", subcore_axis_name="s"),
        compiler_params=pltpu.CompilerParams(needs_layout_passes=False),
        out_type=jax.ShapeDtypeStruct((_NW, _HBINS), jnp.float32),
        scratch_types=[
            pltpu.VMEM((_CH,), jnp.int32),
            pltpu.VMEM((_CH,), jnp.int32),
            pltpu.VMEM((_CH,), jnp.int32),
            pltpu.VMEM((_CH,), jnp.int32),
            pltpu.VMEM((_CH,), jnp.int32),
            pltpu.VMEM((_CH,), jnp.int32),
            pltpu.VMEM((_HBINS,), jnp.float32),
            pltpu.SemaphoreType.DMA,
        ],
    )


def _gate_body(partials_ref, table_ref, w1_ref, w2_ref, slot_ref, out_ref):
    f32 = jnp.float32
    dot = lambda a, b: lax.dot_general(
        a, b, (((1,), (0,)), ((), ())),
        precision=lax.Precision.HIGHEST, preferred_element_type=f32)
    dot_t = lambda a, b: lax.dot_general(
        a, b, (((1,), (1,)), ((), ())),
        precision=lax.Precision.HIGHEST, preferred_element_type=f32)

    counts = jnp.sum(partials_ref[...], axis=0)
    slot = slot_ref[...]
    t = jnp.zeros((3 * _B, _D), f32)
    c_s = jnp.zeros((4 * _B, _D), f32)
    for g in range(_VP // 128):
        tab_g = table_ref[g]
        t = t + dot(counts[:, g, :], tab_g)
        oh_g = (lax.broadcasted_iota(jnp.int32, (4 * _B, 128), 1)
                + g * 128 == slot)
        c_s = c_s + dot(oh_g.astype(f32), tab_g)
    t_q = t[0:_B]
    t_sv = t[_B:2 * _B] + t[2 * _B:3 * _B]
    mq = dot_t(t_q, w1_ref[...])
    msv = dot_t(t_sv, w2_ref[...])

    rep = (lax.broadcasted_iota(jnp.int32, (4 * _B, _B), 0) // 4
           == lax.broadcasted_iota(jnp.int32, (4 * _B, _B), 1))
    repf = rep.astype(f32)
    mq4 = dot(repf, mq)
    msv4 = dot(repf, msv)

    sig = lambda x: 1.0 / (1.0 + jnp.exp(-x))
    out_ref[...] = c_s + sig(c_s * mq4) + sig(c_s * msv4)


_tc_gate = pl.pallas_call(
    _gate_body,
    out_shape=jax.ShapeDtypeStruct((4 * _B, _D), jnp.float32),
)


def kernel(ontology_emb, w1, w2, acts_request_tokens, acts_request_seg,
           acts_slot_tokens, acts_slot_seg, acts_value_tokens, acts_value_seg,
           slot_names):
    partials = _make_sc_hist()(acts_request_tokens, acts_request_seg,
                        acts_slot_tokens, acts_slot_seg,
                        acts_value_tokens, acts_value_seg,
                        jnp.zeros((_HBINS,), jnp.float32))
    v, d = ontology_emb.shape
    table_pad = jnp.pad(ontology_emb, ((0, _VP - v), (0, 0)))
    gates = _tc_gate(partials.reshape(_NW, 3 * _B, _VP // 128, 128),
                     table_pad.reshape(_VP // 128, 128, _D),
                     w1, w2, slot_names.reshape(4 * _B, 1))
    return gates.reshape(_B, 4, _D)


# speedup vs baseline: 1.3158x; 1.3158x over previous
import functools

import jax
import jax.numpy as jnp
from jax import lax
from jax.experimental import pallas as pl
from jax.experimental.pallas import tpu as pltpu
from jax.experimental.pallas import tpu_sc as plsc

_B = 16
_T = 16384
_D = 512
_V = 1000
_VP = 1024
_NW = 32
_CH = _T // _NW


def _hist_body(tq_hbm, sq_hbm, ts_hbm, ss_hbm, tv_hbm, sv_hbm, z_hbm, out_hbm,
               tq_v, sq_v, ts_v, ss_v, tv_v, sv_v, hist_v, sem):
    wid = lax.axis_index("s") * 2 + lax.axis_index("c")
    base = wid * _CH

    cps = [
        pltpu.async_copy(z_hbm, hist_v, sem),
        pltpu.async_copy(tq_hbm.at[pl.ds(base, _CH)], tq_v, sem),
        pltpu.async_copy(sq_hbm.at[pl.ds(base, _CH)], sq_v, sem),
        pltpu.async_copy(ts_hbm.at[pl.ds(base, _CH)], ts_v, sem),
        pltpu.async_copy(ss_hbm.at[pl.ds(base, _CH)], ss_v, sem),
        pltpu.async_copy(tv_hbm.at[pl.ds(base, _CH)], tv_v, sem),
        pltpu.async_copy(sv_hbm.at[pl.ds(base, _CH)], sv_v, sem),
    ]
    for c in cps:
        c.wait()

    ones16 = jnp.ones((16,), jnp.float32)
    for s, (tok_v, seg_v) in enumerate(
            ((tq_v, sq_v), (ts_v, ss_v), (tv_v, sv_v))):
        for i in range(_CH // 16):
            tok = tok_v[pl.ds(i * 16, 16)]
            seg = seg_v[pl.ds(i * 16, 16)]
            plsc.addupdate_scatter(
                hist_v,
                [seg + s * _B, lax.shift_right_logical(tok, 7), tok & 127],
                ones16)

    pltpu.sync_copy(hist_v, out_hbm.at[wid])


@functools.cache
def _make_sc_hist():
    return pl.kernel(
        _hist_body,
        mesh=plsc.VectorSubcoreMesh(core_axis_name="c", subcore_axis_name="s"),
        compiler_params=pltpu.CompilerParams(needs_layout_passes=False),
        out_type=jax.ShapeDtypeStruct((_NW, 3 * _B, _VP // 128, 128),
                                      jnp.float32),
        scratch_types=[
            pltpu.VMEM((_CH,), jnp.int32),
            pltpu.VMEM((_CH,), jnp.int32),
            pltpu.VMEM((_CH,), jnp.int32),
            pltpu.VMEM((_CH,), jnp.int32),
            pltpu.VMEM((_CH,), jnp.int32),
            pltpu.VMEM((_CH,), jnp.int32),
            pltpu.VMEM((3 * _B, _VP // 128, 128), jnp.float32),
            pltpu.SemaphoreType.DMA,
        ],
    )


def _gate_body(partials_ref, table_ref, w1_ref, w2_ref, slot_ref, out_ref):
    f32 = jnp.float32
    dot = lambda a, b: lax.dot_general(
        a, b, (((1,), (0,)), ((), ())),
        precision=lax.Precision.HIGHEST, preferred_element_type=f32)
    dot_t = lambda a, b: lax.dot_general(
        a, b, (((1,), (1,)), ((), ())),
        precision=lax.Precision.HIGHEST, preferred_element_type=f32)

    counts = jnp.sum(partials_ref[...], axis=0)
    slot = slot_ref[...]
    t = jnp.zeros((3 * _B, _D), f32)
    c_s = jnp.zeros((4 * _B, _D), f32)
    for g in range(_VP // 128):
        k = min(128, _V - g * 128)
        tab_g = table_ref[pl.ds(g * 128, k), :]
        t = t + dot(counts[:, g, 0:k], tab_g)
        oh_g = (lax.broadcasted_iota(jnp.int32, (4 * _B, k), 1)
                + g * 128 == slot)
        c_s = c_s + dot(oh_g.astype(f32), tab_g)
    t_q = t[0:_B]
    t_sv = t[_B:2 * _B] + t[2 * _B:3 * _B]
    mq = dot_t(t_q, w1_ref[...])
    msv = dot_t(t_sv, w2_ref[...])

    rep = (lax.broadcasted_iota(jnp.int32, (4 * _B, _B), 0) // 4
           == lax.broadcasted_iota(jnp.int32, (4 * _B, _B), 1))
    repf = rep.astype(f32)
    mq4 = dot(repf, mq)
    msv4 = dot(repf, msv)

    sig = lambda x: 1.0 / (1.0 + jnp.exp(-x))
    g = c_s + sig(c_s * mq4) + sig(c_s * msv4)
    out_ref[...] = g.reshape(_B, 4, _D)


_tc_gate = pl.pallas_call(
    _gate_body,
    out_shape=jax.ShapeDtypeStruct((_B, 4, _D), jnp.float32),
)


def kernel(ontology_emb, w1, w2, acts_request_tokens, acts_request_seg,
           acts_slot_tokens, acts_slot_seg, acts_value_tokens, acts_value_seg,
           slot_names):
    partials = _make_sc_hist()(
        acts_request_tokens, acts_request_seg,
        acts_slot_tokens, acts_slot_seg,
        acts_value_tokens, acts_value_seg,
        jnp.zeros((3 * _B, _VP // 128, 128), jnp.float32))
    return _tc_gate(partials, ontology_emb, w1, w2,
                    slot_names.reshape(4 * _B, 1))


# speedup vs baseline: 1.4250x; 1.0830x over previous
import functools

import jax
import jax.numpy as jnp
from jax import lax
from jax.experimental import pallas as pl
from jax.experimental.pallas import tpu as pltpu
from jax.experimental.pallas import tpu_sc as plsc

_B = 16
_T = 16384
_D = 512
_V = 1000
_VP = 1024
_NW = 32
_CH = _T // _NW


def _hist_body(tq_hbm, sq_hbm, ts_hbm, ss_hbm, tv_hbm, sv_hbm, out_hbm,
               tq_v, sq_v, ts_v, ss_v, tv_v, sv_v, hist_v, sem):
    wid = lax.axis_index("s") * 2 + lax.axis_index("c")
    base = wid * _CH

    cps = [
        pltpu.async_copy(tq_hbm.at[pl.ds(base, _CH)], tq_v, sem),
        pltpu.async_copy(sq_hbm.at[pl.ds(base, _CH)], sq_v, sem),
        pltpu.async_copy(ts_hbm.at[pl.ds(base, _CH)], ts_v, sem),
        pltpu.async_copy(ss_hbm.at[pl.ds(base, _CH)], ss_v, sem),
        pltpu.async_copy(tv_hbm.at[pl.ds(base, _CH)], tv_v, sem),
        pltpu.async_copy(sv_hbm.at[pl.ds(base, _CH)], sv_v, sem),
    ]
    zeros16 = jnp.zeros((16,), jnp.float32)

    def _zero(r, c):
        for j in range(_VP // 128):
            for k in range(8):
                hist_v[r, j, pl.ds(k * 16, 16)] = zeros16
        return c

    lax.fori_loop(0, 3 * _B, _zero, 0)
    for c in cps:
        c.wait()

    ones16 = jnp.ones((16,), jnp.float32)
    for s, (tok_v, seg_v) in enumerate(
            ((tq_v, sq_v), (ts_v, ss_v), (tv_v, sv_v))):
        for i in range(_CH // 16):
            tok = tok_v[pl.ds(i * 16, 16)]
            seg = seg_v[pl.ds(i * 16, 16)]
            plsc.addupdate_scatter(
                hist_v,
                [seg + s * _B, lax.shift_right_logical(tok, 7), tok & 127],
                ones16)

    pltpu.sync_copy(hist_v, out_hbm.at[wid])


@functools.cache
def _make_sc_hist():
    return pl.kernel(
        _hist_body,
        mesh=plsc.VectorSubcoreMesh(core_axis_name="c", subcore_axis_name="s"),
        compiler_params=pltpu.CompilerParams(needs_layout_passes=False),
        out_type=jax.ShapeDtypeStruct((_NW, 3 * _B, _VP // 128, 128),
                                      jnp.float32),
        scratch_types=[
            pltpu.VMEM((_CH,), jnp.int32),
            pltpu.VMEM((_CH,), jnp.int32),
            pltpu.VMEM((_CH,), jnp.int32),
            pltpu.VMEM((_CH,), jnp.int32),
            pltpu.VMEM((_CH,), jnp.int32),
            pltpu.VMEM((_CH,), jnp.int32),
            pltpu.VMEM((3 * _B, _VP // 128, 128), jnp.float32),
            pltpu.SemaphoreType.DMA,
        ],
    )


def _gate_body(partials_hbm, table_ref, w1_ref, w2_ref, slot_ref, out_ref,
               pbuf, psem):
    f32 = jnp.float32
    cp = pltpu.make_async_copy(partials_hbm, pbuf, psem)
    cp.start()
    cp.wait()
    partials_ref = pbuf
    dot = lambda a, b: lax.dot_general(
        a, b, (((1,), (0,)), ((), ())),
        precision=lax.Precision.HIGHEST, preferred_element_type=f32)
    dot_t = lambda a, b: lax.dot_general(
        a, b, (((1,), (1,)), ((), ())),
        precision=lax.Precision.HIGHEST, preferred_element_type=f32)

    counts = jnp.sum(partials_ref[...], axis=0)
    slot = slot_ref[...]
    t = jnp.zeros((3 * _B, _D), f32)
    c_s = jnp.zeros((4 * _B, _D), f32)
    for g in range(_VP // 128):
        k = min(128, _V - g * 128)
        tab_g = table_ref[pl.ds(g * 128, k), :]
        t = t + dot(counts[:, g, 0:k], tab_g)
        oh_g = (lax.broadcasted_iota(jnp.int32, (4 * _B, k), 1)
                + g * 128 == slot)
        c_s = c_s + dot(oh_g.astype(f32), tab_g)
    t_q = t[0:_B]
    t_sv = t[_B:2 * _B] + t[2 * _B:3 * _B]
    mq = dot_t(t_q, w1_ref[...])
    msv = dot_t(t_sv, w2_ref[...])

    rep = (lax.broadcasted_iota(jnp.int32, (4 * _B, _B), 0) // 4
           == lax.broadcasted_iota(jnp.int32, (4 * _B, _B), 1))
    repf = rep.astype(f32)
    mq4 = dot(repf, mq)
    msv4 = dot(repf, msv)

    sig = lambda x: 1.0 / (1.0 + jnp.exp(-x))
    g = c_s + sig(c_s * mq4) + sig(c_s * msv4)
    out_ref[...] = g.reshape(_B, 4, _D)


_tc_gate = pl.pallas_call(
    _gate_body,
    in_specs=[
        pl.BlockSpec(memory_space=pl.ANY),
        pl.BlockSpec(memory_space=pltpu.VMEM),
        pl.BlockSpec(memory_space=pltpu.VMEM),
        pl.BlockSpec(memory_space=pltpu.VMEM),
        pl.BlockSpec(memory_space=pltpu.VMEM),
    ],
    scratch_shapes=[
        pltpu.VMEM((_NW, 3 * _B, _VP // 128, 128), jnp.float32),
        pltpu.SemaphoreType.DMA,
    ],
    out_shape=jax.ShapeDtypeStruct((_B, 4, _D), jnp.float32),
)


def kernel(ontology_emb, w1, w2, acts_request_tokens, acts_request_seg,
           acts_slot_tokens, acts_slot_seg, acts_value_tokens, acts_value_seg,
           slot_names):
    partials = _make_sc_hist()(
        acts_request_tokens, acts_request_seg,
        acts_slot_tokens, acts_slot_seg,
        acts_value_tokens, acts_value_seg)
    return _tc_gate(partials, ontology_emb, w1, w2,
                    slot_names.reshape(4 * _B, 1))


# speedup vs baseline: 1.4443x; 1.0136x over previous
import functools

import jax
import jax.numpy as jnp
from jax import lax
from jax.experimental import pallas as pl
from jax.experimental.pallas import tpu as pltpu
from jax.experimental.pallas import tpu_sc as plsc

_B = 16
_T = 16384
_D = 512
_V = 1000
_VP = 1024
_NW = 32
_CH = _T // _NW


def _hist_body(tq_hbm, sq_hbm, ts_hbm, ss_hbm, tv_hbm, sv_hbm, out_hbm,
               tq_v, sq_v, ts_v, ss_v, tv_v, sv_v, hist_v, sem):
    wid = lax.axis_index("s") * 2 + lax.axis_index("c")
    base = wid * _CH

    cps = [
        pltpu.async_copy(tq_hbm.at[pl.ds(base, _CH)], tq_v, sem),
        pltpu.async_copy(sq_hbm.at[pl.ds(base, _CH)], sq_v, sem),
        pltpu.async_copy(ts_hbm.at[pl.ds(base, _CH)], ts_v, sem),
        pltpu.async_copy(ss_hbm.at[pl.ds(base, _CH)], ss_v, sem),
        pltpu.async_copy(tv_hbm.at[pl.ds(base, _CH)], tv_v, sem),
        pltpu.async_copy(sv_hbm.at[pl.ds(base, _CH)], sv_v, sem),
    ]
    zeros16 = jnp.zeros((16,), jnp.float32)

    def _zero(r, c):
        for j in range(_VP // 128):
            for k in range(8):
                hist_v[r, j, pl.ds(k * 16, 16)] = zeros16
        return c

    lax.fori_loop(0, 3 * _B, _zero, 0)
    for c in cps:
        c.wait()

    ones16 = jnp.ones((16,), jnp.float32)
    outs = []
    for s, (tok_v, seg_v) in enumerate(
            ((tq_v, sq_v), (ts_v, ss_v), (tv_v, sv_v))):
        for i in range(_CH // 16):
            tok = tok_v[pl.ds(i * 16, 16)]
            seg = seg_v[pl.ds(i * 16, 16)]
            plsc.addupdate_scatter(
                hist_v,
                [seg + s * _B, lax.shift_right_logical(tok, 7), tok & 127],
                ones16)
        outs.append(pltpu.async_copy(
            hist_v.at[pl.ds(s * _B, _B)],
            out_hbm.at[wid, pl.ds(s * _B, _B)], sem))
    for c in outs:
        c.wait()


@functools.cache
def _make_sc_hist():
    return pl.kernel(
        _hist_body,
        mesh=plsc.VectorSubcoreMesh(core_axis_name="c", subcore_axis_name="s"),
        compiler_params=pltpu.CompilerParams(needs_layout_passes=False),
        out_type=jax.ShapeDtypeStruct((_NW, 3 * _B, _VP // 128, 128),
                                      jnp.float32),
        scratch_types=[
            pltpu.VMEM((_CH,), jnp.int32),
            pltpu.VMEM((_CH,), jnp.int32),
            pltpu.VMEM((_CH,), jnp.int32),
            pltpu.VMEM((_CH,), jnp.int32),
            pltpu.VMEM((_CH,), jnp.int32),
            pltpu.VMEM((_CH,), jnp.int32),
            pltpu.VMEM((3 * _B, _VP // 128, 128), jnp.float32),
            pltpu.SemaphoreType.DMA,
        ],
    )


def _gate_body(partials_hbm, table_ref, w1_ref, w2_ref, slot_ref, out_ref,
               pbuf, psem):
    f32 = jnp.float32
    cp = pltpu.make_async_copy(partials_hbm, pbuf, psem)
    cp.start()
    cp.wait()
    partials_ref = pbuf
    dot = lambda a, b: lax.dot_general(
        a, b, (((1,), (0,)), ((), ())),
        precision=lax.Precision.HIGHEST, preferred_element_type=f32)
    dot_t = lambda a, b: lax.dot_general(
        a, b, (((1,), (1,)), ((), ())),
        precision=lax.Precision.HIGHEST, preferred_element_type=f32)

    counts = jnp.sum(partials_ref[...], axis=0)
    slot = slot_ref[...]
    t = jnp.zeros((3 * _B, _D), f32)
    c_s = jnp.zeros((4 * _B, _D), f32)
    for g in range(_VP // 128):
        k = min(128, _V - g * 128)
        tab_g = table_ref[pl.ds(g * 128, k), :]
        t = t + dot(counts[:, g, 0:k], tab_g)
        oh_g = (lax.broadcasted_iota(jnp.int32, (4 * _B, k), 1)
                + g * 128 == slot)
        c_s = c_s + dot(oh_g.astype(f32), tab_g)
    t_q = t[0:_B]
    t_sv = t[_B:2 * _B] + t[2 * _B:3 * _B]
    mq = dot_t(t_q, w1_ref[...])
    msv = dot_t(t_sv, w2_ref[...])

    rep = (lax.broadcasted_iota(jnp.int32, (4 * _B, _B), 0) // 4
           == lax.broadcasted_iota(jnp.int32, (4 * _B, _B), 1))
    repf = rep.astype(f32)
    mq4 = dot(repf, mq)
    msv4 = dot(repf, msv)

    sig = lambda x: 1.0 / (1.0 + jnp.exp(-x))
    g = c_s + sig(c_s * mq4) + sig(c_s * msv4)
    out_ref[...] = g.reshape(_B, 4, _D)


_tc_gate = pl.pallas_call(
    _gate_body,
    in_specs=[
        pl.BlockSpec(memory_space=pl.ANY),
        pl.BlockSpec(memory_space=pltpu.VMEM),
        pl.BlockSpec(memory_space=pltpu.VMEM),
        pl.BlockSpec(memory_space=pltpu.VMEM),
        pl.BlockSpec(memory_space=pltpu.VMEM),
    ],
    scratch_shapes=[
        pltpu.VMEM((_NW, 3 * _B, _VP // 128, 128), jnp.float32),
        pltpu.SemaphoreType.DMA,
    ],
    out_shape=jax.ShapeDtypeStruct((_B, 4, _D), jnp.float32),
)


def kernel(ontology_emb, w1, w2, acts_request_tokens, acts_request_seg,
           acts_slot_tokens, acts_slot_seg, acts_value_tokens, acts_value_seg,
           slot_names):
    partials = _make_sc_hist()(
        acts_request_tokens, acts_request_seg,
        acts_slot_tokens, acts_slot_seg,
        acts_value_tokens, acts_value_seg)
    return _tc_gate(partials, ontology_emb, w1, w2,
                    slot_names.reshape(4 * _B, 1))


# speedup vs baseline: 1.4570x; 1.0088x over previous
import functools

import jax
import jax.numpy as jnp
from jax import lax
from jax.experimental import pallas as pl
from jax.experimental.pallas import tpu as pltpu
from jax.experimental.pallas import tpu_sc as plsc

_B = 16
_T = 16384
_D = 512
_V = 1000
_VP = 1024
_NW = 32
_CH = _T // _NW


def _hist_body(tq_hbm, sq_hbm, ts_hbm, ss_hbm, tv_hbm, sv_hbm, out_hbm,
               tq_v, sq_v, ts_v, ss_v, tv_v, sv_v, hist_v, sem):
    wid = lax.axis_index("s") * 2 + lax.axis_index("c")
    base = wid * _CH

    cps = [
        pltpu.async_copy(tq_hbm.at[pl.ds(base, _CH)], tq_v, sem),
        pltpu.async_copy(sq_hbm.at[pl.ds(base, _CH)], sq_v, sem),
        pltpu.async_copy(ts_hbm.at[pl.ds(base, _CH)], ts_v, sem),
        pltpu.async_copy(ss_hbm.at[pl.ds(base, _CH)], ss_v, sem),
        pltpu.async_copy(tv_hbm.at[pl.ds(base, _CH)], tv_v, sem),
        pltpu.async_copy(sv_hbm.at[pl.ds(base, _CH)], sv_v, sem),
    ]
    zeros16 = jnp.zeros((16,), jnp.float32)

    def _zero(r, c):
        for j in range(_VP // 128):
            for k in range(8):
                hist_v[r, j, pl.ds(k * 16, 16)] = zeros16
        return c

    lax.fori_loop(0, 3 * _B, _zero, 0)
    for c in cps:
        c.wait()

    ones16 = jnp.ones((16,), jnp.float32)
    outs = []
    for s, (tok_v, seg_v) in enumerate(
            ((tq_v, sq_v), (ts_v, ss_v), (tv_v, sv_v))):
        for i in range(_CH // 16):
            tok = tok_v[pl.ds(i * 16, 16)]
            seg = seg_v[pl.ds(i * 16, 16)]
            plsc.addupdate_scatter(
                hist_v,
                [seg + s * _B, lax.shift_right_logical(tok, 7), tok & 127],
                ones16)
        outs.append(pltpu.async_copy(
            hist_v.at[pl.ds(s * _B, _B)],
            out_hbm.at[wid, pl.ds(s * _B, _B)], sem))
    for c in outs:
        c.wait()


@functools.cache
def _make_sc_hist():
    return pl.kernel(
        _hist_body,
        mesh=plsc.VectorSubcoreMesh(core_axis_name="c", subcore_axis_name="s"),
        compiler_params=pltpu.CompilerParams(needs_layout_passes=False),
        out_type=jax.ShapeDtypeStruct((_NW, 3 * _B, _VP // 128, 128),
                                      jnp.float32),
        scratch_types=[
            pltpu.VMEM((_CH,), jnp.int32),
            pltpu.VMEM((_CH,), jnp.int32),
            pltpu.VMEM((_CH,), jnp.int32),
            pltpu.VMEM((_CH,), jnp.int32),
            pltpu.VMEM((_CH,), jnp.int32),
            pltpu.VMEM((_CH,), jnp.int32),
            pltpu.VMEM((3 * _B, _VP // 128, 128), jnp.float32),
            pltpu.SemaphoreType.DMA,
        ],
    )


def _gate_body(partials_hbm, table_ref, w1_ref, w2_ref, slot_ref, out_ref,
               pbuf, psem):
    f32 = jnp.float32
    nc = 4
    w = _NW // nc
    cps = [pltpu.make_async_copy(partials_hbm.at[pl.ds(c * w, w)],
                                 pbuf.at[pl.ds(c * w, w)],
                                 psem.at[c])
           for c in range(nc)]
    for cp in cps:
        cp.start()
    counts = jnp.zeros((3 * _B, _VP // 128, 128), f32)
    for c in range(nc):
        cps[c].wait()
        counts = counts + jnp.sum(pbuf[pl.ds(c * w, w)], axis=0)
    dot = lambda a, b: lax.dot_general(
        a, b, (((1,), (0,)), ((), ())),
        precision=lax.Precision.HIGHEST, preferred_element_type=f32)
    dot_t = lambda a, b: lax.dot_general(
        a, b, (((1,), (1,)), ((), ())),
        precision=lax.Precision.HIGHEST, preferred_element_type=f32)

    slot = slot_ref[...]
    t = jnp.zeros((3 * _B, _D), f32)
    c_s = jnp.zeros((4 * _B, _D), f32)
    for g in range(_VP // 128):
        k = min(128, _V - g * 128)
        tab_g = table_ref[pl.ds(g * 128, k), :]
        t = t + dot(counts[:, g, 0:k], tab_g)
        oh_g = (lax.broadcasted_iota(jnp.int32, (4 * _B, k), 1)
                + g * 128 == slot)
        c_s = c_s + dot(oh_g.astype(f32), tab_g)
    t_q = t[0:_B]
    t_sv = t[_B:2 * _B] + t[2 * _B:3 * _B]
    mq = dot_t(t_q, w1_ref[...])
    msv = dot_t(t_sv, w2_ref[...])

    rep = (lax.broadcasted_iota(jnp.int32, (4 * _B, _B), 0) // 4
           == lax.broadcasted_iota(jnp.int32, (4 * _B, _B), 1))
    repf = rep.astype(f32)
    mq4 = dot(repf, mq)
    msv4 = dot(repf, msv)

    sig = lambda x: 1.0 / (1.0 + jnp.exp(-x))
    g = c_s + sig(c_s * mq4) + sig(c_s * msv4)
    out_ref[...] = g.reshape(_B, 4, _D)


_tc_gate = pl.pallas_call(
    _gate_body,
    in_specs=[
        pl.BlockSpec(memory_space=pl.ANY),
        pl.BlockSpec(memory_space=pltpu.VMEM),
        pl.BlockSpec(memory_space=pltpu.VMEM),
        pl.BlockSpec(memory_space=pltpu.VMEM),
        pl.BlockSpec(memory_space=pltpu.VMEM),
    ],
    scratch_shapes=[
        pltpu.VMEM((_NW, 3 * _B, _VP // 128, 128), jnp.float32),
        pltpu.SemaphoreType.DMA((4,)),
    ],
    out_shape=jax.ShapeDtypeStruct((_B, 4, _D), jnp.float32),
)


def kernel(ontology_emb, w1, w2, acts_request_tokens, acts_request_seg,
           acts_slot_tokens, acts_slot_seg, acts_value_tokens, acts_value_seg,
           slot_names):
    partials = _make_sc_hist()(
        acts_request_tokens, acts_request_seg,
        acts_slot_tokens, acts_slot_seg,
        acts_value_tokens, acts_value_seg)
    return _tc_gate(partials, ontology_emb, w1, w2,
                    slot_names.reshape(4 * _B, 1))
